# initial kernel scaffold (unmeasured)
import jax
import jax.numpy as jnp
from jax import lax
from jax.experimental import pallas as pl
from jax.experimental.pallas import tpu as pltpu


def kernel(
    x,
):
    def body(*refs):
        pass

    out_shape = jax.ShapeDtypeStruct(..., jnp.float32)
    return pl.pallas_call(body, out_shape=out_shape)(...)


# baseline (device time: 17417 ns/iter reference)
import jax
import jax.numpy as jnp
from jax import lax
from jax.experimental import pallas as pl
from jax.experimental.pallas import tpu as pltpu


def kernel(x):
    m, n = x.shape

    def body(x_ref, out_ref, send_buf, recv_buf, send_sem, recv_sem):
        my_x = lax.axis_index("x")
        my_y = lax.axis_index("y")
        my_z = lax.axis_index("z")
        peer = (my_x, 1 - my_y, my_z)

        barrier_sem = pltpu.get_barrier_semaphore()
        pl.semaphore_signal(
            barrier_sem, inc=1,
            device_id=peer, device_id_type=pl.DeviceIdType.MESH,
        )
        pl.semaphore_wait(barrier_sem, 1)

        send_buf[...] = x_ref[...].astype(jnp.bfloat16)

        rdma = pltpu.make_async_remote_copy(
            src_ref=send_buf,
            dst_ref=recv_buf,
            send_sem=send_sem,
            recv_sem=recv_sem,
            device_id=peer,
            device_id_type=pl.DeviceIdType.MESH,
        )
        rdma.start()
        rdma.wait()

        out_ref[...] = send_buf[...] + recv_buf[...]

    return pl.pallas_call(
        body,
        out_shape=jax.ShapeDtypeStruct((m, n), jnp.bfloat16),
        in_specs=[pl.BlockSpec(memory_space=pltpu.VMEM)],
        out_specs=pl.BlockSpec(memory_space=pltpu.VMEM),
        scratch_shapes=[
            pltpu.VMEM((m, n), jnp.bfloat16),
            pltpu.VMEM((m, n), jnp.bfloat16),
            pltpu.SemaphoreType.DMA,
            pltpu.SemaphoreType.DMA,
        ],
        compiler_params=pltpu.CompilerParams(collective_id=0),
    )(x)


# device time: 16961 ns/iter; 1.0269x vs baseline; 1.0269x over previous
import jax
import jax.numpy as jnp
from jax import lax
from jax.experimental import pallas as pl
from jax.experimental.pallas import tpu as pltpu

C = 4


def kernel(x):
    m, n = x.shape
    half = m // 2
    rc = half // C

    def body(x_ref, out_ref, sendy, ybuf, xrecv_buf,
             ysend, yrecv, xsend, xrecv, ack_y, ack_x):
        my_x = lax.axis_index("x")
        my_y = lax.axis_index("y")
        my_z = lax.axis_index("z")
        yp = (my_x, 1 - my_y, my_z)
        xn = (1 - my_x, my_y, my_z)
        base = my_x * half

        barrier_sem = pltpu.get_barrier_semaphore()
        for nbr in (yp, xn):
            pl.semaphore_signal(
                barrier_sem, inc=1,
                device_id=nbr, device_id_type=pl.DeviceIdType.MESH,
            )
        pl.semaphore_wait(barrier_sem, 2)

        rdma_y = []
        for i in range(C):
            sendy[i] = x_ref[pl.ds(base + i * rc, rc), :].astype(jnp.bfloat16)
            r = pltpu.make_async_remote_copy(
                src_ref=sendy.at[i],
                dst_ref=ybuf.at[i],
                send_sem=ysend.at[i],
                recv_sem=yrecv.at[i],
                device_id=yp,
                device_id_type=pl.DeviceIdType.MESH,
            )
            r.start()
            rdma_y.append(r)

        rdma_x = []
        for i in range(C):
            rdma_y[i].wait_recv()
            out_ref[pl.ds(base + i * rc, rc), :] = sendy[i] + ybuf[i]
            r = pltpu.make_async_remote_copy(
                src_ref=out_ref.at[pl.ds(base + i * rc, rc), :],
                dst_ref=xrecv_buf.at[i],
                send_sem=xsend.at[i],
                recv_sem=xrecv.at[i],
                device_id=xn,
                device_id_type=pl.DeviceIdType.MESH,
            )
            r.start()
            rdma_x.append(r)

        other = (1 - my_x) * half
        for i in range(C):
            rdma_x[i].wait_recv()
            out_ref[pl.ds(other + i * rc, rc), :] = xrecv_buf[i]

        for i in range(C):
            rdma_y[i].wait_send()
            rdma_x[i].wait_send()

        pl.semaphore_signal(ack_y, inc=1, device_id=yp,
                            device_id_type=pl.DeviceIdType.MESH)
        pl.semaphore_signal(ack_x, inc=1, device_id=xn,
                            device_id_type=pl.DeviceIdType.MESH)
        pl.semaphore_wait(ack_y, 1)
        pl.semaphore_wait(ack_x, 1)

    return pl.pallas_call(
        body,
        out_shape=jax.ShapeDtypeStruct((m, n), jnp.bfloat16),
        in_specs=[pl.BlockSpec(memory_space=pltpu.VMEM)],
        out_specs=pl.BlockSpec(memory_space=pltpu.VMEM),
        scratch_shapes=[
            pltpu.VMEM((C, rc, n), jnp.bfloat16),
            pltpu.VMEM((C, rc, n), jnp.bfloat16),
            pltpu.VMEM((C, rc, n), jnp.bfloat16),
            pltpu.SemaphoreType.DMA((C,)),
            pltpu.SemaphoreType.DMA((C,)),
            pltpu.SemaphoreType.DMA((C,)),
            pltpu.SemaphoreType.DMA((C,)),
            pltpu.SemaphoreType.REGULAR,
            pltpu.SemaphoreType.REGULAR,
        ],
        compiler_params=pltpu.CompilerParams(collective_id=0),
    )(x)


# device time: 16317 ns/iter; 1.0674x vs baseline; 1.0395x over previous
import jax
import jax.numpy as jnp
from jax import lax
from jax.experimental import pallas as pl
from jax.experimental.pallas import tpu as pltpu

C = 8


def kernel(x):
    m, n = x.shape
    half = m // 2
    rc = half // C

    def body(x_hbm, out_ref, xl, sendy, ybuf, xrecv_buf,
             lsem, ysend, yrecv, xsend, xrecv, sync_x):
        my_x = lax.axis_index("x")
        my_y = lax.axis_index("y")
        my_z = lax.axis_index("z")
        yp = (my_x, 1 - my_y, my_z)
        xn = (1 - my_x, my_y, my_z)
        base = my_x * half

        barrier_sem = pltpu.get_barrier_semaphore()
        pl.semaphore_signal(barrier_sem, inc=1, device_id=yp,
                            device_id_type=pl.DeviceIdType.MESH)
        pl.semaphore_signal(sync_x, inc=1, device_id=xn,
                            device_id_type=pl.DeviceIdType.MESH)

        lcp = []
        for i in range(C):
            c = pltpu.make_async_copy(
                x_hbm.at[pl.ds(base + i * rc, rc), :], xl.at[i], lsem.at[i])
            c.start()
            lcp.append(c)

        pl.semaphore_wait(barrier_sem, 1)

        rdma_y = []
        for i in range(C):
            lcp[i].wait()
            sendy[i] = xl[i].astype(jnp.bfloat16)
            r = pltpu.make_async_remote_copy(
                src_ref=sendy.at[i], dst_ref=ybuf.at[i],
                send_sem=ysend.at[i], recv_sem=yrecv.at[i],
                device_id=yp, device_id_type=pl.DeviceIdType.MESH,
            )
            r.start()
            rdma_y.append(r)

        pl.semaphore_wait(sync_x, 1)

        rdma_x = []
        for i in range(C):
            rdma_y[i].wait_recv()
            out_ref[pl.ds(base + i * rc, rc), :] = sendy[i] + ybuf[i]
            r = pltpu.make_async_remote_copy(
                src_ref=out_ref.at[pl.ds(base + i * rc, rc), :],
                dst_ref=xrecv_buf.at[i],
                send_sem=xsend.at[i], recv_sem=xrecv.at[i],
                device_id=xn, device_id_type=pl.DeviceIdType.MESH,
            )
            r.start()
            rdma_x.append(r)

        other = (1 - my_x) * half
        for i in range(C):
            rdma_x[i].wait_recv()
            out_ref[pl.ds(other + i * rc, rc), :] = xrecv_buf[i]

        for i in range(C):
            rdma_y[i].wait_send()
            rdma_x[i].wait_send()

    return pl.pallas_call(
        body,
        out_shape=jax.ShapeDtypeStruct((m, n), jnp.bfloat16),
        in_specs=[pl.BlockSpec(memory_space=pl.ANY)],
        out_specs=pl.BlockSpec(memory_space=pltpu.VMEM),
        scratch_shapes=[
            pltpu.VMEM((C, rc, n), jnp.float32),
            pltpu.VMEM((C, rc, n), jnp.bfloat16),
            pltpu.VMEM((C, rc, n), jnp.bfloat16),
            pltpu.VMEM((C, rc, n), jnp.bfloat16),
            pltpu.SemaphoreType.DMA((C,)),
            pltpu.SemaphoreType.DMA((C,)),
            pltpu.SemaphoreType.DMA((C,)),
            pltpu.SemaphoreType.DMA((C,)),
            pltpu.SemaphoreType.DMA((C,)),
            pltpu.SemaphoreType.REGULAR,
        ],
        compiler_params=pltpu.CompilerParams(collective_id=0),
    )(x)
